# baseline (speedup 1.0000x reference)
import functools

import jax
import jax.numpy as jnp
from jax import lax
from jax.experimental import pallas as pl
from jax.experimental.pallas import tpu as pltpu
from jax.experimental.pallas import tpu_sc as plsc

N = 10000
NR = 12
NB = 8
D = 128
E = 320000
BATCH = 64
L = 32
M = 256

NC = 2
NS = 16
NW = NC * NS
CK = 128
NCHUNK = 80
EPAD = NW * NCHUNK * CK

SEGP = 120064
SEG_PT = SEGP // NS
AGGP = 10240
ROWS_PT = AGGP // NS

NIDS = BATCH * L + M
NID_CHUNKS = NIDS // CK

COLS = N * D
CB = 128000

_sc_mesh = plsc.VectorSubcoreMesh(core_axis_name="c", subcore_axis_name="s")


BN = 2000


def _weight_body(comp_ref, basis_ref, out_ref):
    r = pl.program_id(1)
    crow = comp_ref[pl.ds(r, 1), :]
    cb = [jnp.broadcast_to(crow[:, b:b + 1], (8, D)) for b in range(NB)]

    def _g(g, c):
        sl = pl.ds(g * 8, 8)
        acc = cb[0] * basis_ref[0, sl, :]
        for b in range(1, NB):
            acc += cb[b] * basis_ref[b, sl, :]
        out_ref[sl, :] = acc
        return c
    lax.fori_loop(0, BN // 8, _g, 0, unroll=4)


def _compute_weight(comp, basis):
    return pl.pallas_call(
        _weight_body,
        grid=(N // BN, NR),
        in_specs=[
            pl.BlockSpec((NR, NB), lambda i, r: (0, 0)),
            pl.BlockSpec((NB, BN, D), lambda i, r: (0, i, 0)),
        ],
        out_specs=pl.BlockSpec((BN, D), lambda i, r: (r * (N // BN) + i, 0)),
        out_shape=jax.ShapeDtypeStruct((NR * N, D), jnp.float32),
    )(comp, basis)


@functools.partial(
    pl.kernel,
    out_type=(jax.ShapeDtypeStruct((SEGP,), jnp.float32),
              jax.ShapeDtypeStruct((SEGP,), jnp.float32)),
    mesh=_sc_mesh,
    scratch_types=[
        pltpu.VMEM((NCHUNK, CK), jnp.int32),
        pltpu.VMEM((SEG_PT,), jnp.float32),
        pltpu.VMEM((CK,), jnp.float32),
        pltpu.VMEM_SHARED((SEGP,), jnp.float32),
    ],
)
def _hist_kernel(seg_hbm, cnt0_hbm, cnt1_hbm, idx_v, zbuf_v, ones_v, cnt_sh):
    cid = lax.axis_index("c")
    sid = lax.axis_index("s")
    wid = cid * NS + sid

    def _zero(i, c):
        zbuf_v[pl.ds(i * 16, 16)] = jnp.zeros((16,), jnp.float32)
        return c
    lax.fori_loop(0, SEG_PT // 16, _zero, 0)
    for k in range(CK // 16):
        ones_v[pl.ds(k * 16, 16)] = jnp.ones((16,), jnp.float32)
    pltpu.sync_copy(zbuf_v, cnt_sh.at[pl.ds(sid * SEG_PT, SEG_PT)])
    plsc.subcore_barrier()

    pltpu.sync_copy(seg_hbm.at[wid], idx_v)

    def _scatter(j, c):
        pltpu.sync_copy(ones_v, cnt_sh.at[idx_v.at[j]], add=True)
        return c
    lax.fori_loop(0, NCHUNK, _scatter, 0)
    plsc.subcore_barrier()

    pltpu.sync_copy(cnt_sh.at[pl.ds(sid * SEG_PT, SEG_PT)], zbuf_v)

    @pl.when(cid == 0)
    def _():
        pltpu.sync_copy(zbuf_v, cnt0_hbm.at[pl.ds(sid * SEG_PT, SEG_PT)])

    @pl.when(cid == 1)
    def _():
        pltpu.sync_copy(zbuf_v, cnt1_hbm.at[pl.ds(sid * SEG_PT, SEG_PT)])


@functools.partial(
    pl.kernel,
    out_type=(jax.ShapeDtypeStruct((NIDS, D), jnp.float32),
              jax.ShapeDtypeStruct((NIDS, D), jnp.float32)),
    mesh=_sc_mesh,
    scratch_types=[
        pltpu.VMEM((2, CK), jnp.int32),
        pltpu.VMEM((2, CK), jnp.int32),
        pltpu.VMEM((2, CK), jnp.int32),
        pltpu.VMEM((2, CK), jnp.int32),
        pltpu.VMEM((2, CK, D), jnp.float32),
        pltpu.VMEM((2, CK), jnp.float32),
        pltpu.VMEM((2, CK), jnp.float32),
        pltpu.VMEM((8, D), jnp.float32),
        pltpu.VMEM_SHARED((AGGP, D), jnp.float32),
        pltpu.SemaphoreType.DMA,
        pltpu.SemaphoreType.DMA,
        pltpu.SemaphoreType.DMA,
        pltpu.SemaphoreType.DMA,
        pltpu.SemaphoreType.DMA,
        pltpu.SemaphoreType.DMA,
    ],
)
def _msg_kernel(weight_hbm, cnt0_hbm, cnt1_hbm, rowi_hbm, segi_hbm, dsti_hbm,
                ids_hbm, root_hbm,
                reps0_hbm, reps1_hbm,
                rowi_v, segi_v, dsti_v, dscat_v, rows_v,
                c0_v, c1_v, zb_v, agg_sh,
                gsem0, gsem1, isem0, isem1, ssem0, ssem1):
    cid = lax.axis_index("c")
    sid = lax.axis_index("s")
    wid = cid * NS + sid
    row0 = sid * ROWS_PT
    gsem = (gsem0, gsem1)
    isem = (isem0, isem1)
    ssem = (ssem0, ssem1)

    def _issue_idx(j, b):
        pltpu.async_copy(rowi_hbm.at[wid, j], rowi_v.at[b], isem[b])
        pltpu.async_copy(segi_hbm.at[wid, j], segi_v.at[b], isem[b])
        pltpu.async_copy(dsti_hbm.at[wid, j], dsti_v.at[b], isem[b])

    def _wait_idx(b):
        pltpu.make_async_copy(rowi_hbm.at[wid, 0], rowi_v.at[b], isem[b]).wait()
        pltpu.make_async_copy(segi_hbm.at[wid, 0], segi_v.at[b], isem[b]).wait()
        pltpu.make_async_copy(dsti_hbm.at[wid, 0], dsti_v.at[b], isem[b]).wait()

    def _issue_gathers(b):
        pltpu.async_copy(weight_hbm.at[rowi_v.at[b]], rows_v.at[b], gsem[b])
        pltpu.async_copy(cnt0_hbm.at[segi_v.at[b]], c0_v.at[b], gsem[b])
        pltpu.async_copy(cnt1_hbm.at[segi_v.at[b]], c1_v.at[b], gsem[b])

    def _wait_gathers(b):
        pltpu.make_async_copy(weight_hbm.at[rowi_v.at[b]], rows_v.at[b],
                              gsem[b]).wait()
        pltpu.make_async_copy(cnt0_hbm.at[segi_v.at[b]], c0_v.at[b],
                              gsem[b]).wait()
        pltpu.make_async_copy(cnt1_hbm.at[segi_v.at[b]], c1_v.at[b],
                              gsem[b]).wait()

    def _issue_scatter(b):
        pltpu.async_copy(rows_v.at[b], agg_sh.at[dscat_v.at[b]], ssem[b],
                         add=True)

    def _wait_scatter(b):
        pltpu.make_async_copy(rows_v.at[b], agg_sh.at[dscat_v.at[b]],
                              ssem[b]).wait()

    _issue_idx(0, 0)
    _issue_idx(1, 1)

    def _zrow(r, c):
        for k in range(D // 16):
            zb_v[r, pl.ds(k * 16, 16)] = jnp.zeros((16,), jnp.float32)
        return c
    lax.fori_loop(0, 8, _zrow, 0)

    def _zcopy(k, c):
        pltpu.sync_copy(zb_v, agg_sh.at[pl.ds(row0 + k * 8, 8)])
        return c
    lax.fori_loop(0, ROWS_PT // 8, _zcopy, 0)

    _wait_idx(0)
    _issue_gathers(0)
    plsc.subcore_barrier()

    def _outer(t, c):
        for b in range(2):
            j = t * 2 + b
            nb = 1 - b
            _wait_gathers(b)

            @pl.when(j >= 1)
            def _():
                _wait_scatter(nb)

            @pl.when(j + 1 < NCHUNK)
            def _():
                _wait_idx(nb)
                _issue_gathers(nb)

            for k in range(CK // 16):
                sl = pl.ds(k * 16, 16)
                dscat_v[b, sl] = dsti_v[b, sl]

            def _grp(g, cc):
                sl16 = pl.ds(g * 16, 16)
                sv = 1.0 / jnp.maximum(c0_v[b, sl16] + c1_v[b, sl16], 1.0)
                for l in range(16):
                    spl = jax.lax.broadcast(sv[l], (16,))
                    e = g * 16 + l
                    for k in range(D // 16):
                        sl = pl.ds(k * 16, 16)
                        rows_v[b, e, sl] = rows_v[b, e, sl] * spl
                return cc
            lax.fori_loop(0, CK // 16, _grp, 0)
            _issue_scatter(b)

            @pl.when(j + 2 < NCHUNK)
            def _():
                _issue_idx(j + 2, b)
        return c
    lax.fori_loop(0, NCHUNK // 2, _outer, 0)
    _wait_scatter((NCHUNK - 1) % 2)
    plsc.subcore_barrier()

    def _gather_reps(ci):
        pltpu.sync_copy(ids_hbm.at[ci], rowi_v.at[0])
        cpg = pltpu.async_copy(agg_sh.at[rowi_v.at[0]], rows_v.at[0], gsem0)

        @pl.when(cid == 0)
        def _():
            cpr = pltpu.async_copy(root_hbm.at[rowi_v.at[0]], rows_v.at[1],
                                   gsem1)
            cpg.wait()
            cpr.wait()

            def _row(e, c):
                for k in range(D // 16):
                    sl = pl.ds(k * 16, 16)
                    rows_v[0, e, sl] = rows_v[0, e, sl] + rows_v[1, e, sl]
                return c
            lax.fori_loop(0, CK, _row, 0)
            pltpu.sync_copy(rows_v.at[0], reps0_hbm.at[pl.ds(ci * CK, CK)])

        @pl.when(cid == 1)
        def _():
            cpg.wait()
            pltpu.sync_copy(rows_v.at[0], reps1_hbm.at[pl.ds(ci * CK, CK)])

    _gather_reps(sid)

    @pl.when(sid < NID_CHUNKS - NS)
    def _():
        _gather_reps(NS + sid)


def _head_body(reps0_ref, reps1_ref, bias_ref, g_ref, gt_ref, aa_ref, ab_ref,
               kw1_ref, kb1_ref, kw2_ref, kb2_ref,
               ew1_ref, eb1_ref, ew2_ref, eb2_ref,
               user_ref, ent_out_ref):
    reps = reps0_ref[...] + reps1_ref[...]
    rb = reps[:BATCH * L] + bias_ref[...]
    t = jnp.tanh(jax.lax.dot(rb, aa_ref[...],
                             preferred_element_type=jnp.float32))
    e = jnp.sum(t * ab_ref[...], axis=1, keepdims=True)
    x = jnp.exp(e)
    denom = jax.lax.dot(g_ref[...], x, preferred_element_type=jnp.float32)
    dpe = jax.lax.dot(gt_ref[...], denom, preferred_element_type=jnp.float32)
    attn = x / dpe
    ur = jax.lax.dot(g_ref[...], attn * rb,
                     preferred_element_type=jnp.float32)
    h = jnp.maximum(jax.lax.dot(ur, kw1_ref[...],
                                preferred_element_type=jnp.float32)
                    + kb1_ref[...], 0.0)
    user_ref[...] = jax.lax.dot(h, kw2_ref[...],
                                preferred_element_type=jnp.float32) + kb2_ref[...]

    re = reps[BATCH * L:] + bias_ref[...]
    he = jnp.maximum(jax.lax.dot(re, ew1_ref[...],
                                 preferred_element_type=jnp.float32)
                     + eb1_ref[...], 0.0)
    ent_out_ref[...] = jax.lax.dot(he, ew2_ref[...],
                                   preferred_element_type=jnp.float32) + eb2_ref[...]


def _head(reps0, reps1, bias, g, gt, attn_a, ab,
          kw1, kb1, kw2, kb2, ew1, eb1, ew2, eb2):
    return pl.pallas_call(
        _head_body,
        out_shape=(jax.ShapeDtypeStruct((BATCH, D), jnp.float32),
                   jax.ShapeDtypeStruct((M, D), jnp.float32)),
    )(reps0, reps1, bias, g, gt, attn_a, ab,
      kw1, kb1, kw2, kb2, ew1, eb1, ew2, eb2)


def kernel(edge_index, edge_type, context_entities, entity_ids,
           basis, comp, root, bias, attn_a, attn_b,
           kg_fc1_w, kg_fc1_b, kg_fc2_w, kg_fc2_b,
           ekg_fc1_w, ekg_fc1_b, ekg_fc2_w, ekg_fc2_b):
    src = edge_index[0].astype(jnp.int32)
    dst = edge_index[1].astype(jnp.int32)
    et = edge_type.astype(jnp.int32)
    pad = EPAD - E

    parange = jnp.arange(pad, dtype=jnp.int32)
    row_idx = jnp.concatenate([et * N + src, parange % N])
    seg_idx = jnp.concatenate([dst * NR + et,
                               N * NR + parange % (SEGP - N * NR)])
    dst_idx = jnp.concatenate([dst, N + parange % (AGGP - N)])
    def _chunked(x):
        return x.reshape(NCHUNK, NW, CK).transpose(1, 0, 2)
    row3 = _chunked(row_idx)
    seg3 = _chunked(seg_idx)
    dst3 = _chunked(dst_idx)

    weight2d = _compute_weight(comp, basis)

    ids = jnp.concatenate([context_entities.astype(jnp.int32).reshape(-1),
                           entity_ids.astype(jnp.int32)]).reshape(NID_CHUNKS, CK)
    cnt0, cnt1 = _hist_kernel(seg3)
    reps0, reps1 = _msg_kernel(weight2d, cnt0, cnt1,
                               row3, seg3, dst3, ids, root)

    g = (jnp.arange(BATCH * L, dtype=jnp.int32)[None, :] // L
         == jnp.arange(BATCH, dtype=jnp.int32)[:, None]).astype(jnp.float32)

    user_proj, ent_proj = _head(
        reps0, reps1, bias.reshape(1, D), g, g.T,
        attn_a, attn_b.reshape(1, D),
        kg_fc1_w, kg_fc1_b.reshape(1, D), kg_fc2_w, kg_fc2_b.reshape(1, D),
        ekg_fc1_w, ekg_fc1_b.reshape(1, D), ekg_fc2_w, ekg_fc2_b.reshape(1, D))
    return (user_proj, ent_proj)


# speedup vs baseline: 1.0545x; 1.0545x over previous
import functools

import jax
import jax.numpy as jnp
from jax import lax
from jax.experimental import pallas as pl
from jax.experimental.pallas import tpu as pltpu
from jax.experimental.pallas import tpu_sc as plsc

N = 10000
NR = 12
NB = 8
D = 128
E = 320000
BATCH = 64
L = 32
M = 256

NC = 2
NS = 16
NW = NC * NS
CK = 128
NCHUNK = 80
EPAD = NW * NCHUNK * CK

SEGP = 120064
SEG_PT = SEGP // NS
AGGP = 10240
ROWS_PT = AGGP // NS

NIDS = BATCH * L + M
NID_CHUNKS = NIDS // CK

COLS = N * D
CB = 128000

_sc_mesh = plsc.VectorSubcoreMesh(core_axis_name="c", subcore_axis_name="s")


BN = 2000


def _weight_body(comp_ref, basis_ref, out_ref):
    r = pl.program_id(1)
    crow = comp_ref[pl.ds(r, 1), :]
    cb = [jnp.broadcast_to(crow[:, b:b + 1], (8, D)) for b in range(NB)]

    def _g(g, c):
        sl = pl.ds(g * 8, 8)
        acc = cb[0] * basis_ref[0, sl, :]
        for b in range(1, NB):
            acc += cb[b] * basis_ref[b, sl, :]
        out_ref[sl, :] = acc
        return c
    lax.fori_loop(0, BN // 8, _g, 0, unroll=4)


def _compute_weight(comp, basis):
    return pl.pallas_call(
        _weight_body,
        grid=(N // BN, NR),
        in_specs=[
            pl.BlockSpec((NR, NB), lambda i, r: (0, 0)),
            pl.BlockSpec((NB, BN, D), lambda i, r: (0, i, 0)),
        ],
        out_specs=pl.BlockSpec((BN, D), lambda i, r: (r * (N // BN) + i, 0)),
        out_shape=jax.ShapeDtypeStruct((NR * N, D), jnp.float32),
    )(comp, basis)


@functools.partial(
    pl.kernel,
    out_type=(jax.ShapeDtypeStruct((SEGP,), jnp.float32),
              jax.ShapeDtypeStruct((SEGP,), jnp.float32)),
    mesh=_sc_mesh,
    scratch_types=[
        pltpu.VMEM((NCHUNK, CK), jnp.int32),
        pltpu.VMEM((SEG_PT,), jnp.float32),
        pltpu.VMEM((CK,), jnp.float32),
        pltpu.VMEM_SHARED((SEGP,), jnp.float32),
    ],
)
def _hist_kernel(seg_hbm, cnt0_hbm, cnt1_hbm, idx_v, zbuf_v, ones_v, cnt_sh):
    cid = lax.axis_index("c")
    sid = lax.axis_index("s")
    wid = cid * NS + sid

    def _zero(i, c):
        zbuf_v[pl.ds(i * 16, 16)] = jnp.zeros((16,), jnp.float32)
        return c
    lax.fori_loop(0, SEG_PT // 16, _zero, 0)
    for k in range(CK // 16):
        ones_v[pl.ds(k * 16, 16)] = jnp.ones((16,), jnp.float32)
    pltpu.sync_copy(zbuf_v, cnt_sh.at[pl.ds(sid * SEG_PT, SEG_PT)])
    plsc.subcore_barrier()

    pltpu.sync_copy(seg_hbm.at[wid], idx_v)

    def _scatter(j, c):
        pltpu.sync_copy(ones_v, cnt_sh.at[idx_v.at[j]], add=True)
        return c
    lax.fori_loop(0, NCHUNK, _scatter, 0)
    plsc.subcore_barrier()

    pltpu.sync_copy(cnt_sh.at[pl.ds(sid * SEG_PT, SEG_PT)], zbuf_v)

    @pl.when(cid == 0)
    def _():
        pltpu.sync_copy(zbuf_v, cnt0_hbm.at[pl.ds(sid * SEG_PT, SEG_PT)])

    @pl.when(cid == 1)
    def _():
        pltpu.sync_copy(zbuf_v, cnt1_hbm.at[pl.ds(sid * SEG_PT, SEG_PT)])


@functools.partial(
    pl.kernel,
    out_type=(jax.ShapeDtypeStruct((NIDS, D), jnp.float32),
              jax.ShapeDtypeStruct((NIDS, D), jnp.float32)),
    mesh=_sc_mesh,
    scratch_types=[
        pltpu.VMEM((2, CK), jnp.int32),
        pltpu.VMEM((2, CK), jnp.int32),
        pltpu.VMEM((2, CK), jnp.int32),
        pltpu.VMEM((2, CK), jnp.int32),
        pltpu.VMEM((2, CK, D), jnp.float32),
        pltpu.VMEM((2, CK), jnp.float32),
        pltpu.VMEM((2, CK), jnp.float32),
        pltpu.VMEM((8, D), jnp.float32),
        pltpu.VMEM_SHARED((AGGP, D), jnp.float32),
        pltpu.SemaphoreType.DMA,
        pltpu.SemaphoreType.DMA,
        pltpu.SemaphoreType.DMA,
        pltpu.SemaphoreType.DMA,
        pltpu.SemaphoreType.DMA,
        pltpu.SemaphoreType.DMA,
    ],
)
def _msg_kernel(weight_hbm, cnt0_hbm, cnt1_hbm, rowi_hbm, segi_hbm, dsti_hbm,
                ids_hbm, root_hbm,
                reps0_hbm, reps1_hbm,
                rowi_v, segi_v, dsti_v, dscat_v, rows_v,
                c0_v, c1_v, zb_v, agg_sh,
                gsem0, gsem1, isem0, isem1, ssem0, ssem1):
    cid = lax.axis_index("c")
    sid = lax.axis_index("s")
    wid = cid * NS + sid
    row0 = sid * ROWS_PT
    gsem = (gsem0, gsem1)
    isem = (isem0, isem1)
    ssem = (ssem0, ssem1)

    def _issue_idx(j, b):
        pltpu.async_copy(rowi_hbm.at[wid, j], rowi_v.at[b], isem[b])
        pltpu.async_copy(segi_hbm.at[wid, j], segi_v.at[b], isem[b])
        pltpu.async_copy(dsti_hbm.at[wid, j], dsti_v.at[b], isem[b])

    def _wait_idx(b):
        pltpu.make_async_copy(rowi_hbm.at[wid, 0], rowi_v.at[b], isem[b]).wait()
        pltpu.make_async_copy(segi_hbm.at[wid, 0], segi_v.at[b], isem[b]).wait()
        pltpu.make_async_copy(dsti_hbm.at[wid, 0], dsti_v.at[b], isem[b]).wait()

    def _issue_gathers(b):
        pltpu.async_copy(weight_hbm.at[rowi_v.at[b]], rows_v.at[b], gsem[b])
        pltpu.async_copy(cnt0_hbm.at[segi_v.at[b]], c0_v.at[b], gsem[b])
        pltpu.async_copy(cnt1_hbm.at[segi_v.at[b]], c1_v.at[b], gsem[b])

    def _wait_gathers(b):
        pltpu.make_async_copy(weight_hbm.at[rowi_v.at[b]], rows_v.at[b],
                              gsem[b]).wait()
        pltpu.make_async_copy(cnt0_hbm.at[segi_v.at[b]], c0_v.at[b],
                              gsem[b]).wait()
        pltpu.make_async_copy(cnt1_hbm.at[segi_v.at[b]], c1_v.at[b],
                              gsem[b]).wait()

    def _issue_scatter(b):
        pltpu.async_copy(rows_v.at[b], agg_sh.at[dscat_v.at[b]], ssem[b],
                         add=True)

    def _wait_scatter(b):
        pltpu.make_async_copy(rows_v.at[b], agg_sh.at[dscat_v.at[b]],
                              ssem[b]).wait()

    _issue_idx(0, 0)
    _issue_idx(1, 1)

    def _zrow(r, c):
        for k in range(D // 16):
            zb_v[r, pl.ds(k * 16, 16)] = jnp.zeros((16,), jnp.float32)
        return c
    lax.fori_loop(0, 8, _zrow, 0)

    def _zcopy(k, c):
        pltpu.sync_copy(zb_v, agg_sh.at[pl.ds(row0 + k * 8, 8)])
        return c
    lax.fori_loop(0, ROWS_PT // 8, _zcopy, 0)

    _wait_idx(0)
    _issue_gathers(0)
    plsc.subcore_barrier()

    def _outer(t, c):
        for b in range(2):
            j = t * 2 + b
            nb = 1 - b
            _wait_gathers(b)

            @pl.when(j >= 1)
            def _():
                _wait_scatter(nb)

            @pl.when(j + 1 < NCHUNK)
            def _():
                _wait_idx(nb)
                _issue_gathers(nb)

            for k in range(CK // 16):
                sl = pl.ds(k * 16, 16)
                dscat_v[b, sl] = dsti_v[b, sl]

            def _grp(g, cc):
                sl16 = pl.ds(g * 16, 16)
                sv = 1.0 / jnp.maximum(c0_v[b, sl16] + c1_v[b, sl16], 1.0)
                for l in range(16):
                    spl = jax.lax.broadcast(sv[l], (16,))
                    e = g * 16 + l
                    for k in range(D // 16):
                        sl = pl.ds(k * 16, 16)
                        rows_v[b, e, sl] = rows_v[b, e, sl] * spl
                return cc
            lax.fori_loop(0, CK // 16, _grp, 0)
            _issue_scatter(b)

            @pl.when(j + 2 < NCHUNK)
            def _():
                _issue_idx(j + 2, b)
        return c
    lax.fori_loop(0, NCHUNK // 2, _outer, 0)
    _wait_scatter((NCHUNK - 1) % 2)
    plsc.subcore_barrier()

    def _gather_reps(ci):
        pltpu.sync_copy(ids_hbm.at[ci], rowi_v.at[0])
        cpg = pltpu.async_copy(agg_sh.at[rowi_v.at[0]], rows_v.at[0], gsem0)

        @pl.when(cid == 0)
        def _():
            cpr = pltpu.async_copy(root_hbm.at[rowi_v.at[0]], rows_v.at[1],
                                   gsem1)
            cpg.wait()
            cpr.wait()

            def _row(e, c):
                for k in range(D // 16):
                    sl = pl.ds(k * 16, 16)
                    rows_v[0, e, sl] = rows_v[0, e, sl] + rows_v[1, e, sl]
                return c
            lax.fori_loop(0, CK, _row, 0)
            pltpu.sync_copy(rows_v.at[0], reps0_hbm.at[pl.ds(ci * CK, CK)])

        @pl.when(cid == 1)
        def _():
            cpg.wait()
            pltpu.sync_copy(rows_v.at[0], reps1_hbm.at[pl.ds(ci * CK, CK)])

    _gather_reps(sid)

    @pl.when(sid < NID_CHUNKS - NS)
    def _():
        _gather_reps(NS + sid)


def _head_body(reps0_ref, reps1_ref, bias_ref, g_ref, gt_ref, aa_ref, ab_ref,
               kw1_ref, kb1_ref, kw2_ref, kb2_ref,
               ew1_ref, eb1_ref, ew2_ref, eb2_ref,
               user_ref, ent_out_ref):
    reps = reps0_ref[...] + reps1_ref[...]
    rb = reps[:BATCH * L] + bias_ref[...]
    t = jnp.tanh(jax.lax.dot(rb, aa_ref[...],
                             preferred_element_type=jnp.float32))
    e = jnp.sum(t * ab_ref[...], axis=1, keepdims=True)
    x = jnp.exp(e)
    denom = jax.lax.dot(g_ref[...], x, preferred_element_type=jnp.float32)
    dpe = jax.lax.dot(gt_ref[...], denom, preferred_element_type=jnp.float32)
    attn = x / dpe
    ur = jax.lax.dot(g_ref[...], attn * rb,
                     preferred_element_type=jnp.float32)
    h = jnp.maximum(jax.lax.dot(ur, kw1_ref[...],
                                preferred_element_type=jnp.float32)
                    + kb1_ref[...], 0.0)
    user_ref[...] = jax.lax.dot(h, kw2_ref[...],
                                preferred_element_type=jnp.float32) + kb2_ref[...]

    re = reps[BATCH * L:] + bias_ref[...]
    he = jnp.maximum(jax.lax.dot(re, ew1_ref[...],
                                 preferred_element_type=jnp.float32)
                     + eb1_ref[...], 0.0)
    ent_out_ref[...] = jax.lax.dot(he, ew2_ref[...],
                                   preferred_element_type=jnp.float32) + eb2_ref[...]


def _head(reps0, reps1, bias, g, gt, attn_a, ab,
          kw1, kb1, kw2, kb2, ew1, eb1, ew2, eb2):
    return pl.pallas_call(
        _head_body,
        out_shape=(jax.ShapeDtypeStruct((BATCH, D), jnp.float32),
                   jax.ShapeDtypeStruct((M, D), jnp.float32)),
    )(reps0, reps1, bias, g, gt, attn_a, ab,
      kw1, kb1, kw2, kb2, ew1, eb1, ew2, eb2)


def kernel(edge_index, edge_type, context_entities, entity_ids,
           basis, comp, root, bias, attn_a, attn_b,
           kg_fc1_w, kg_fc1_b, kg_fc2_w, kg_fc2_b,
           ekg_fc1_w, ekg_fc1_b, ekg_fc2_w, ekg_fc2_b):
    src = edge_index[0].astype(jnp.int32)
    dst = edge_index[1].astype(jnp.int32)
    et = edge_type.astype(jnp.int32)
    pad = EPAD - E

    p1 = pad // 2
    pa = jnp.arange(p1, dtype=jnp.int32)
    pb = jnp.arange(pad - p1, dtype=jnp.int32)
    row_idx = jnp.concatenate([pa % N, et * N + src, pb % N])
    seg_idx = jnp.concatenate([N * NR + pa % (SEGP - N * NR),
                               dst * NR + et,
                               N * NR + pb % (SEGP - N * NR)])
    dst_idx = jnp.concatenate([N + pa % (AGGP - N), dst,
                               N + pb % (AGGP - N)])
    row3 = row_idx.reshape(NW, NCHUNK, CK)
    seg3 = seg_idx.reshape(NW, NCHUNK, CK)
    dst3 = dst_idx.reshape(NW, NCHUNK, CK)

    weight2d = _compute_weight(comp, basis)

    ids = jnp.concatenate([context_entities.astype(jnp.int32).reshape(-1),
                           entity_ids.astype(jnp.int32)]).reshape(NID_CHUNKS, CK)
    cnt0, cnt1 = _hist_kernel(seg3)
    reps0, reps1 = _msg_kernel(weight2d, cnt0, cnt1,
                               row3, seg3, dst3, ids, root)

    g = (jnp.arange(BATCH * L, dtype=jnp.int32)[None, :] // L
         == jnp.arange(BATCH, dtype=jnp.int32)[:, None]).astype(jnp.float32)

    user_proj, ent_proj = _head(
        reps0, reps1, bias.reshape(1, D), g, g.T,
        attn_a, attn_b.reshape(1, D),
        kg_fc1_w, kg_fc1_b.reshape(1, D), kg_fc2_w, kg_fc2_b.reshape(1, D),
        ekg_fc1_w, ekg_fc1_b.reshape(1, D), ekg_fc2_w, ekg_fc2_b.reshape(1, D))
    return (user_proj, ent_proj)


# speedup vs baseline: 1.0550x; 1.0004x over previous
import functools

import jax
import jax.numpy as jnp
from jax import lax
from jax.experimental import pallas as pl
from jax.experimental.pallas import tpu as pltpu
from jax.experimental.pallas import tpu_sc as plsc

N = 10000
NR = 12
NB = 8
D = 128
E = 320000
BATCH = 64
L = 32
M = 256

NC = 2
NS = 16
NW = NC * NS
CK = 128
NCHUNK = 80
EPAD = NW * NCHUNK * CK

SEGP = 120064
SEG_PT = SEGP // NS
AGGP = 10240
ROWS_PT = AGGP // NS

NIDS = BATCH * L + M
NID_CHUNKS = NIDS // CK


_sc_mesh = plsc.VectorSubcoreMesh(core_axis_name="c", subcore_axis_name="s")


BN = 2000


def _weight_body(comp_ref, basis_ref, out_ref):
    r = pl.program_id(1)
    crow = comp_ref[pl.ds(r, 1), :]
    cb = [jnp.broadcast_to(crow[:, b:b + 1], (8, D)) for b in range(NB)]

    def _g(g, c):
        sl = pl.ds(g * 8, 8)
        acc = cb[0] * basis_ref[0, sl, :]
        for b in range(1, NB):
            acc += cb[b] * basis_ref[b, sl, :]
        out_ref[sl, :] = acc
        return c
    lax.fori_loop(0, BN // 8, _g, 0, unroll=4)


def _compute_weight(comp, basis):
    return pl.pallas_call(
        _weight_body,
        grid=(N // BN, NR),
        in_specs=[
            pl.BlockSpec((NR, NB), lambda i, r: (0, 0)),
            pl.BlockSpec((NB, BN, D), lambda i, r: (0, i, 0)),
        ],
        out_specs=pl.BlockSpec((BN, D), lambda i, r: (r * (N // BN) + i, 0)),
        out_shape=jax.ShapeDtypeStruct((NR * N, D), jnp.float32),
    )(comp, basis)


@functools.partial(
    pl.kernel,
    out_type=(jax.ShapeDtypeStruct((SEGP,), jnp.float32),
              jax.ShapeDtypeStruct((SEGP,), jnp.float32)),
    mesh=_sc_mesh,
    scratch_types=[
        pltpu.VMEM((NCHUNK, CK), jnp.int32),
        pltpu.VMEM((SEG_PT,), jnp.float32),
        pltpu.VMEM((CK,), jnp.float32),
        pltpu.VMEM_SHARED((SEGP,), jnp.float32),
    ],
)
def _hist_kernel(seg_hbm, cnt0_hbm, cnt1_hbm, idx_v, zbuf_v, ones_v, cnt_sh):
    cid = lax.axis_index("c")
    sid = lax.axis_index("s")
    wid = cid * NS + sid

    def _zero(i, c):
        zbuf_v[pl.ds(i * 16, 16)] = jnp.zeros((16,), jnp.float32)
        return c
    lax.fori_loop(0, SEG_PT // 16, _zero, 0)
    for k in range(CK // 16):
        ones_v[pl.ds(k * 16, 16)] = jnp.ones((16,), jnp.float32)
    pltpu.sync_copy(zbuf_v, cnt_sh.at[pl.ds(sid * SEG_PT, SEG_PT)])
    plsc.subcore_barrier()

    pltpu.sync_copy(seg_hbm.at[wid], idx_v)

    def _scatter(j, c):
        pltpu.sync_copy(ones_v, cnt_sh.at[idx_v.at[j]], add=True)
        return c
    lax.fori_loop(0, NCHUNK, _scatter, 0)
    plsc.subcore_barrier()

    pltpu.sync_copy(cnt_sh.at[pl.ds(sid * SEG_PT, SEG_PT)], zbuf_v)

    @pl.when(cid == 0)
    def _():
        pltpu.sync_copy(zbuf_v, cnt0_hbm.at[pl.ds(sid * SEG_PT, SEG_PT)])

    @pl.when(cid == 1)
    def _():
        pltpu.sync_copy(zbuf_v, cnt1_hbm.at[pl.ds(sid * SEG_PT, SEG_PT)])


@functools.partial(
    pl.kernel,
    out_type=(jax.ShapeDtypeStruct((NIDS, D), jnp.float32),
              jax.ShapeDtypeStruct((NIDS, D), jnp.float32)),
    mesh=_sc_mesh,
    scratch_types=[
        pltpu.VMEM((2, CK), jnp.int32),
        pltpu.VMEM((2, CK), jnp.int32),
        pltpu.VMEM((2, CK), jnp.int32),
        pltpu.VMEM((2, CK), jnp.int32),
        pltpu.VMEM((2, CK, D), jnp.float32),
        pltpu.VMEM((2, CK), jnp.float32),
        pltpu.VMEM((2, CK), jnp.float32),
        pltpu.VMEM((8, D), jnp.float32),
        pltpu.VMEM_SHARED((AGGP, D), jnp.float32),
        pltpu.SemaphoreType.DMA,
        pltpu.SemaphoreType.DMA,
        pltpu.SemaphoreType.DMA,
        pltpu.SemaphoreType.DMA,
        pltpu.SemaphoreType.DMA,
        pltpu.SemaphoreType.DMA,
    ],
)
def _msg_kernel(weight_hbm, cnt0_hbm, cnt1_hbm, rowi_hbm, segi_hbm, dsti_hbm,
                ids_hbm, root_hbm,
                reps0_hbm, reps1_hbm,
                rowi_v, segi_v, dsti_v, dscat_v, rows_v,
                c0_v, c1_v, zb_v, agg_sh,
                gsem0, gsem1, isem0, isem1, ssem0, ssem1):
    cid = lax.axis_index("c")
    sid = lax.axis_index("s")
    wid = cid * NS + sid
    row0 = sid * ROWS_PT
    gsem = (gsem0, gsem1)
    isem = (isem0, isem1)
    ssem = (ssem0, ssem1)

    def _issue_idx(j, b):
        pltpu.async_copy(rowi_hbm.at[wid, j], rowi_v.at[b], isem[b])
        pltpu.async_copy(segi_hbm.at[wid, j], segi_v.at[b], isem[b])
        pltpu.async_copy(dsti_hbm.at[wid, j], dsti_v.at[b], isem[b])

    def _wait_idx(b):
        pltpu.make_async_copy(rowi_hbm.at[wid, 0], rowi_v.at[b], isem[b]).wait()
        pltpu.make_async_copy(segi_hbm.at[wid, 0], segi_v.at[b], isem[b]).wait()
        pltpu.make_async_copy(dsti_hbm.at[wid, 0], dsti_v.at[b], isem[b]).wait()

    def _issue_gathers(b):
        pltpu.async_copy(weight_hbm.at[rowi_v.at[b]], rows_v.at[b], gsem[b])
        pltpu.async_copy(cnt0_hbm.at[segi_v.at[b]], c0_v.at[b], gsem[b])
        pltpu.async_copy(cnt1_hbm.at[segi_v.at[b]], c1_v.at[b], gsem[b])

    def _wait_gathers(b):
        pltpu.make_async_copy(weight_hbm.at[rowi_v.at[b]], rows_v.at[b],
                              gsem[b]).wait()
        pltpu.make_async_copy(cnt0_hbm.at[segi_v.at[b]], c0_v.at[b],
                              gsem[b]).wait()
        pltpu.make_async_copy(cnt1_hbm.at[segi_v.at[b]], c1_v.at[b],
                              gsem[b]).wait()

    def _issue_scatter(b):
        pltpu.async_copy(rows_v.at[b], agg_sh.at[dscat_v.at[b]], ssem[b],
                         add=True)

    def _wait_scatter(b):
        pltpu.make_async_copy(rows_v.at[b], agg_sh.at[dscat_v.at[b]],
                              ssem[b]).wait()

    _issue_idx(0, 0)
    _issue_idx(1, 1)

    def _zrow(r, c):
        for k in range(D // 16):
            zb_v[r, pl.ds(k * 16, 16)] = jnp.zeros((16,), jnp.float32)
        return c
    lax.fori_loop(0, 8, _zrow, 0)

    def _zcopy(k, c):
        pltpu.sync_copy(zb_v, agg_sh.at[pl.ds(row0 + k * 8, 8)])
        return c
    lax.fori_loop(0, ROWS_PT // 8, _zcopy, 0)

    _wait_idx(0)
    _issue_gathers(0)
    plsc.subcore_barrier()

    def _outer(t, c):
        for b in range(2):
            j = t * 2 + b
            nb = 1 - b
            _wait_gathers(b)

            @pl.when(j >= 1)
            def _():
                _wait_scatter(nb)

            @pl.when(j + 1 < NCHUNK)
            def _():
                _wait_idx(nb)
                _issue_gathers(nb)

            for k in range(CK // 16):
                sl = pl.ds(k * 16, 16)
                dscat_v[b, sl] = dsti_v[b, sl]

            def _grp(g, cc):
                sl16 = pl.ds(g * 16, 16)
                sv = 1.0 / jnp.maximum(c0_v[b, sl16] + c1_v[b, sl16], 1.0)
                for l in range(16):
                    spl = jax.lax.broadcast(sv[l], (16,))
                    e = g * 16 + l
                    for k in range(D // 16):
                        sl = pl.ds(k * 16, 16)
                        rows_v[b, e, sl] = rows_v[b, e, sl] * spl
                return cc
            lax.fori_loop(0, CK // 16, _grp, 0)
            _issue_scatter(b)

            @pl.when(j + 2 < NCHUNK)
            def _():
                _issue_idx(j + 2, b)
        return c
    lax.fori_loop(0, NCHUNK // 2, _outer, 0)
    _wait_scatter((NCHUNK - 1) % 2)
    plsc.subcore_barrier()

    def _gather_reps(ci):
        pltpu.sync_copy(ids_hbm.at[ci], rowi_v.at[0])
        cpg = pltpu.async_copy(agg_sh.at[rowi_v.at[0]], rows_v.at[0], gsem0)

        @pl.when(cid == 0)
        def _():
            cpr = pltpu.async_copy(root_hbm.at[rowi_v.at[0]], rows_v.at[1],
                                   gsem1)
            cpg.wait()
            cpr.wait()

            def _row(e, c):
                for k in range(D // 16):
                    sl = pl.ds(k * 16, 16)
                    rows_v[0, e, sl] = rows_v[0, e, sl] + rows_v[1, e, sl]
                return c
            lax.fori_loop(0, CK, _row, 0)
            pltpu.sync_copy(rows_v.at[0], reps0_hbm.at[pl.ds(ci * CK, CK)])

        @pl.when(cid == 1)
        def _():
            cpg.wait()
            pltpu.sync_copy(rows_v.at[0], reps1_hbm.at[pl.ds(ci * CK, CK)])

    _gather_reps(sid)

    @pl.when(sid < NID_CHUNKS - NS)
    def _():
        _gather_reps(NS + sid)


def _head_body(reps0_ref, reps1_ref, bias_ref, g_ref, gt_ref, aa_ref, ab_ref,
               kw1_ref, kb1_ref, kw2_ref, kb2_ref,
               ew1_ref, eb1_ref, ew2_ref, eb2_ref,
               user_ref, ent_out_ref):
    reps = reps0_ref[...] + reps1_ref[...]
    rb = reps[:BATCH * L] + bias_ref[...]
    t = jnp.tanh(jax.lax.dot(rb, aa_ref[...],
                             preferred_element_type=jnp.float32))
    e = jnp.sum(t * ab_ref[...], axis=1, keepdims=True)
    x = jnp.exp(e)
    denom = jax.lax.dot(g_ref[...], x, preferred_element_type=jnp.float32)
    dpe = jax.lax.dot(gt_ref[...], denom, preferred_element_type=jnp.float32)
    attn = x / dpe
    ur = jax.lax.dot(g_ref[...], attn * rb,
                     preferred_element_type=jnp.float32)
    h = jnp.maximum(jax.lax.dot(ur, kw1_ref[...],
                                preferred_element_type=jnp.float32)
                    + kb1_ref[...], 0.0)
    user_ref[...] = jax.lax.dot(h, kw2_ref[...],
                                preferred_element_type=jnp.float32) + kb2_ref[...]

    re = reps[BATCH * L:] + bias_ref[...]
    he = jnp.maximum(jax.lax.dot(re, ew1_ref[...],
                                 preferred_element_type=jnp.float32)
                     + eb1_ref[...], 0.0)
    ent_out_ref[...] = jax.lax.dot(he, ew2_ref[...],
                                   preferred_element_type=jnp.float32) + eb2_ref[...]


def _head(reps0, reps1, bias, g, gt, attn_a, ab,
          kw1, kb1, kw2, kb2, ew1, eb1, ew2, eb2):
    return pl.pallas_call(
        _head_body,
        out_shape=(jax.ShapeDtypeStruct((BATCH, D), jnp.float32),
                   jax.ShapeDtypeStruct((M, D), jnp.float32)),
    )(reps0, reps1, bias, g, gt, attn_a, ab,
      kw1, kb1, kw2, kb2, ew1, eb1, ew2, eb2)


def kernel(edge_index, edge_type, context_entities, entity_ids,
           basis, comp, root, bias, attn_a, attn_b,
           kg_fc1_w, kg_fc1_b, kg_fc2_w, kg_fc2_b,
           ekg_fc1_w, ekg_fc1_b, ekg_fc2_w, ekg_fc2_b):
    src = edge_index[0].astype(jnp.int32)
    dst = edge_index[1].astype(jnp.int32)
    et = edge_type.astype(jnp.int32)
    pad = EPAD - E

    p1 = pad // 2
    pa = jnp.arange(p1, dtype=jnp.int32)
    pb = jnp.arange(pad - p1, dtype=jnp.int32)
    row_idx = jnp.concatenate([pa % N, et * N + src, pb % N])
    seg_idx = jnp.concatenate([N * NR + pa % (SEGP - N * NR),
                               dst * NR + et,
                               N * NR + pb % (SEGP - N * NR)])
    dst_idx = jnp.concatenate([N + pa % (AGGP - N), dst,
                               N + pb % (AGGP - N)])
    row3 = row_idx.reshape(NW, NCHUNK, CK)
    seg3 = seg_idx.reshape(NW, NCHUNK, CK)
    dst3 = dst_idx.reshape(NW, NCHUNK, CK)

    weight2d = _compute_weight(comp, basis)

    ids = jnp.concatenate([context_entities.astype(jnp.int32).reshape(-1),
                           entity_ids.astype(jnp.int32)]).reshape(NID_CHUNKS, CK)
    cnt0, cnt1 = _hist_kernel(seg3)
    reps0, reps1 = _msg_kernel(weight2d, cnt0, cnt1,
                               row3, seg3, dst3, ids, root)

    g = (jnp.arange(BATCH * L, dtype=jnp.int32)[None, :] // L
         == jnp.arange(BATCH, dtype=jnp.int32)[:, None]).astype(jnp.float32)

    user_proj, ent_proj = _head(
        reps0, reps1, bias.reshape(1, D), g, g.T,
        attn_a, attn_b.reshape(1, D),
        kg_fc1_w, kg_fc1_b.reshape(1, D), kg_fc2_w, kg_fc2_b.reshape(1, D),
        ekg_fc1_w, ekg_fc1_b.reshape(1, D), ekg_fc2_w, ekg_fc2_b.reshape(1, D))
    return (user_proj, ent_proj)
